# baseline (speedup 1.0000x reference)
import functools
import jax
import jax.numpy as jnp
from jax import lax
from jax.experimental import pallas as pl
from jax.experimental.pallas import tpu as pltpu
from jax.experimental.pallas import tpu_sc as plsc

_B, _D, _RANK, _K = 16384, 64, 6, 32
_NW = 32
_RPW = _B // _NW
_NC = 4
_CH = 128
_SCALE = 1.0 / (_RANK ** 0.5)
_INTMIN = -0x80000000


def _bf(v):
    b = lax.bitcast_convert_type(v, jnp.int32)
    b = (b + 0x7FFF + jnp.bitwise_and(lax.shift_right_logical(b, 16), 1))
    b = jnp.bitwise_and(b, -65536)
    return lax.bitcast_convert_type(b, jnp.float32)


def _sc_kernel(x_hbm, wst_hbm, bsel_hbm, ut_hbm, v_hbm, wg_hbm, bg_hbm,
               out_hbm, xbuf, obuf, wst, bsel, ut, vv, wg, bg, kbuf, xrow):
    wid = lax.axis_index("s") * 2 + lax.axis_index("c")
    base = wid * _RPW
    pltpu.sync_copy(wst_hbm, wst)
    pltpu.sync_copy(bsel_hbm, bsel)
    pltpu.sync_copy(ut_hbm, ut)
    pltpu.sync_copy(v_hbm, vv)
    pltpu.sync_copy(wg_hbm, wg)
    pltpu.sync_copy(bg_hbm, bg)

    lane = lax.iota(jnp.int32, 16)
    idx_c = [lane + 16 * c for c in range(_NC)]

    def row_body(r, carry):
        xv = [xbuf[r, pl.ds(16 * c, 16)] for c in range(_NC)]
        xr = [_bf(xv[c]) for c in range(_NC)]
        for c in range(_NC):
            xrow[pl.ds(16 * c, 16)] = xr[c]

        def sel_body(i, z):
            i16 = jnp.full((16,), i, jnp.int32)
            bxi = plsc.load_gather(xrow, [i16])
            return tuple(z[c] + bxi * wst[i, pl.ds(16 * c, 16)]
                         for c in range(_NC))

        z0 = tuple(bsel[pl.ds(16 * c, 16)] for c in range(_NC))
        z = lax.fori_loop(0, _D, sel_body, z0)

        kint = []
        for c in range(_NC):
            b = lax.bitcast_convert_type(z[c], jnp.int32)
            kc = jnp.where(b >= 0, b, _INTMIN - b)
            kbuf[pl.ds(16 * c, 16)] = kc
            kint.append(kc)

        kadj0 = tuple(kint[c] - (idx_c[c] > 0).astype(jnp.int32)
                      for c in range(_NC))
        rank0 = tuple(jnp.zeros((16,), jnp.int32) for _ in range(_NC))

        def rank_body(i, kr):
            kadj, rank = kr
            i16 = jnp.full((16,), i, jnp.int32)
            bki = plsc.load_gather(kbuf, [i16])
            rank = tuple(rank[c] + (bki > kadj[c]).astype(jnp.int32)
                         for c in range(_NC))
            kadj = tuple(kadj[c] + (idx_c[c] == i + 1).astype(jnp.int32)
                         for c in range(_NC))
            return kadj, rank

        _, rank = lax.fori_loop(0, _D, rank_body, (kadj0, rank0))

        xs = [jnp.where(rank[c] < _K, xv[c], 0.0) for c in range(_NC)]
        xsr = [jnp.where(rank[c] < _K, xr[c], 0.0) for c in range(_NC)]

        t = []
        for p in range(_RANK):
            acc = xsr[0] * ut[p, pl.ds(0, 16)]
            for c in range(1, _NC):
                acc = acc + xsr[c] * ut[p, pl.ds(16 * c, 16)]
            t.append(_bf(jnp.broadcast_to(jnp.sum(acc), (16,))))
        cross = []
        for c in range(_NC):
            acc = t[0] * vv[0, pl.ds(16 * c, 16)]
            for p in range(1, _RANK):
                acc = acc + t[p] * vv[p, pl.ds(16 * c, 16)]
            cross.append(acc)
        xi = [xs[c] + _SCALE * (xs[c] * cross[c]) for c in range(_NC)]

        gacc = _bf(xi[0]) * wg[pl.ds(0, 16)]
        for c in range(1, _NC):
            gacc = gacc + _bf(xi[c]) * wg[pl.ds(16 * c, 16)]
        gz = jnp.broadcast_to(jnp.sum(gacc), (16,)) + bg[...]
        g = 1.0 / (1.0 + jnp.exp(-gz))

        for c in range(_NC):
            obuf[r, pl.ds(16 * c, 16)] = g * xi[c] + (1.0 - g) * xs[c]
        return carry

    for ch in range(_RPW // _CH):
        cbase = base + ch * _CH
        pltpu.sync_copy(x_hbm.at[pl.ds(cbase, _CH)], xbuf)
        lax.fori_loop(0, _CH, row_body, 0)
        pltpu.sync_copy(obuf, out_hbm.at[pl.ds(cbase, _CH)])


def kernel(x, W_sel, b_sel, U, V, W_gate, b_gate):
    bf = lambda a: a.astype(jnp.bfloat16).astype(jnp.float32)
    wst = bf(W_sel.T)
    ut = bf(U.T)
    V = bf(V)
    wg = bf(W_gate.reshape(_D))
    bg16 = jnp.broadcast_to(b_gate.reshape(1), (16,))
    mesh = plsc.VectorSubcoreMesh(core_axis_name="c", subcore_axis_name="s")
    f = functools.partial(
        pl.kernel,
        mesh=mesh,
        compiler_params=pltpu.CompilerParams(needs_layout_passes=False),
        out_type=jax.ShapeDtypeStruct((_B, _D), jnp.float32),
        scratch_types=[
            pltpu.VMEM((_CH, _D), jnp.float32),
            pltpu.VMEM((_CH, _D), jnp.float32),
            pltpu.VMEM((_D, _D), jnp.float32),
            pltpu.VMEM((_D,), jnp.float32),
            pltpu.VMEM((_RANK, _D), jnp.float32),
            pltpu.VMEM((_RANK, _D), jnp.float32),
            pltpu.VMEM((_D,), jnp.float32),
            pltpu.VMEM((16,), jnp.float32),
            pltpu.VMEM((_D,), jnp.int32),
            pltpu.VMEM((_D,), jnp.float32),
        ],
    )(_sc_kernel)
    return f(x, wst, b_sel, ut, V, wg, bg16)


# speedup vs baseline: 4.2739x; 4.2739x over previous
import functools
import jax
import jax.numpy as jnp
from jax import lax
from jax.experimental import pallas as pl
from jax.experimental.pallas import tpu as pltpu
from jax.experimental.pallas import tpu_sc as plsc

_B, _D, _RANK, _K = 16384, 64, 6, 32
_SB = 2048
_NW = 32
_RPW = _SB // _NW
_CH = 64
_BM = 1024
_SCALE = 1.0 / (_RANK ** 0.5)
_INTMIN = -0x80000000


def _bf(v):
    b = lax.bitcast_convert_type(v, jnp.int32)
    b = (b + 0x7FFF + jnp.bitwise_and(lax.shift_right_logical(b, 16), 1))
    b = jnp.bitwise_and(b, -65536)
    return lax.bitcast_convert_type(b, jnp.float32)


def _sc_kernel(x_hbm, wst_hbm, bsel_hbm, ut_hbm, v_hbm, wg_hbm, bg_hbm,
               out_hbm, xbuf, obuf, wst, bsel, ut, vv, wg, bg, kbuf, xrow):
    wid = lax.axis_index("s") * 2 + lax.axis_index("c")
    base = wid * _RPW
    pltpu.sync_copy(wst_hbm, wst)
    pltpu.sync_copy(bsel_hbm, bsel)
    pltpu.sync_copy(ut_hbm, ut)
    pltpu.sync_copy(v_hbm, vv)
    pltpu.sync_copy(wg_hbm, wg)
    pltpu.sync_copy(bg_hbm, bg)

    lane = lax.iota(jnp.int32, 16)
    idx_c = [lane + 16 * c for c in range(4)]

    def row_body(r, carry):
        xv = [xbuf[r, pl.ds(16 * c, 16)] for c in range(4)]
        xr = [_bf(xv[c]) for c in range(4)]
        for c in range(4):
            xrow[pl.ds(16 * c, 16)] = xr[c]

        def sel_body(i, z):
            i16 = jnp.full((16,), i, jnp.int32)
            bxi = plsc.load_gather(xrow, [i16])
            return tuple(z[c] + bxi * wst[i, pl.ds(16 * c, 16)]
                         for c in range(4))

        z0 = tuple(bsel[pl.ds(16 * c, 16)] for c in range(4))
        z = lax.fori_loop(0, _D, sel_body, z0)

        kint = []
        for c in range(4):
            b = lax.bitcast_convert_type(z[c], jnp.int32)
            kc = jnp.where(b >= 0, b, _INTMIN - b)
            kbuf[pl.ds(16 * c, 16)] = kc
            kint.append(kc)

        kadj0 = tuple(kint[c] - (idx_c[c] > 0).astype(jnp.int32)
                      for c in range(4))
        rank0 = tuple(jnp.zeros((16,), jnp.int32) for _ in range(4))

        def rank_body(i, kr):
            kadj, rank = kr
            i16 = jnp.full((16,), i, jnp.int32)
            bki = plsc.load_gather(kbuf, [i16])
            rank = tuple(rank[c] + (bki > kadj[c]).astype(jnp.int32)
                         for c in range(4))
            kadj = tuple(kadj[c] + (idx_c[c] == i + 1).astype(jnp.int32)
                         for c in range(4))
            return kadj, rank

        _, rank = lax.fori_loop(0, _D, rank_body, (kadj0, rank0))

        xs = [jnp.where(rank[c] < _K, xv[c], 0.0) for c in range(4)]
        xsr = [jnp.where(rank[c] < _K, xr[c], 0.0) for c in range(4)]

        t = []
        for p in range(_RANK):
            acc = xsr[0] * ut[p, pl.ds(0, 16)]
            for c in range(1, 4):
                acc = acc + xsr[c] * ut[p, pl.ds(16 * c, 16)]
            t.append(_bf(jnp.broadcast_to(jnp.sum(acc), (16,))))
        cross = []
        for c in range(4):
            acc = t[0] * vv[0, pl.ds(16 * c, 16)]
            for p in range(1, _RANK):
                acc = acc + t[p] * vv[p, pl.ds(16 * c, 16)]
            cross.append(acc)
        xi = [xs[c] + _SCALE * (xs[c] * cross[c]) for c in range(4)]

        gacc = _bf(xi[0]) * wg[pl.ds(0, 16)]
        for c in range(1, 4):
            gacc = gacc + _bf(xi[c]) * wg[pl.ds(16 * c, 16)]
        gz = jnp.broadcast_to(jnp.sum(gacc), (16,)) + bg[...]
        g = 1.0 / (1.0 + jnp.exp(-gz))

        for c in range(4):
            obuf[r, pl.ds(16 * c, 16)] = g * xi[c] + (1.0 - g) * xs[c]
        return carry

    for ch in range(_RPW // _CH):
        cbase = base + ch * _CH
        pltpu.sync_copy(x_hbm.at[pl.ds(cbase, _CH)], xbuf)
        lax.fori_loop(0, _CH, row_body, 0)
        pltpu.sync_copy(obuf, out_hbm.at[pl.ds(cbase, _CH)])


def _tc_kernel(x_ref, wsel_ref, bsel_ref, u_ref, v_ref, wg_ref, bg_ref,
               out_ref):
    x = x_ref[...]
    xt = x.T
    zt = jax.lax.dot_general(wsel_ref[...], x, (((1,), (1,)), ((), ())),
                             preferred_element_type=jnp.float32)
    zt = zt + bsel_ref[...]

    row = jax.lax.broadcasted_iota(jnp.int32, (_D, _BM), 0)
    b = zt.view(jnp.int32)
    k = jnp.where(b >= 0, b, jnp.int32(_INTMIN) - b)

    kadj = jnp.where(row > 0, k - 1, k)
    rank = jnp.zeros((_D, _BM), jnp.int32)
    for i in range(_D):
        if i > 0:
            onehot = (row == i).astype(jnp.int32)
            kadj = kadj + onehot
        rank = rank + (kadj[i:i + 1, :] > kadj).astype(jnp.int32)
    xs = jnp.where(rank < _K, xt, 0.0)

    m = jnp.dot(u_ref[...], v_ref[...], preferred_element_type=jnp.float32)
    crosst = jax.lax.dot_general(m, xs, (((0,), (0,)), ((), ())),
                                 preferred_element_type=jnp.float32)
    xi = xs * (1.0 + _SCALE * crosst)

    g = jax.nn.sigmoid(
        jnp.sum(xi * wg_ref[...], axis=0, keepdims=True) + bg_ref[...])
    out_ref[...] = (g * xi + (1.0 - g) * xs).T


def kernel(x, W_sel, b_sel, U, V, W_gate, b_gate):
    bf = lambda a: a.astype(jnp.bfloat16).astype(jnp.float32)
    wst = bf(W_sel.T)
    ut = bf(U.T)
    vr = bf(V)
    wg = bf(W_gate.reshape(_D))
    bg16 = jnp.broadcast_to(b_gate.reshape(1), (16,))
    mesh = plsc.VectorSubcoreMesh(core_axis_name="c", subcore_axis_name="s")
    sc = functools.partial(
        pl.kernel,
        mesh=mesh,
        compiler_params=pltpu.CompilerParams(needs_layout_passes=False),
        out_type=jax.ShapeDtypeStruct((_SB, _D), jnp.float32),
        scratch_types=[
            pltpu.VMEM((_CH, _D), jnp.float32),
            pltpu.VMEM((_CH, _D), jnp.float32),
            pltpu.VMEM((_D, _D), jnp.float32),
            pltpu.VMEM((_D,), jnp.float32),
            pltpu.VMEM((_RANK, _D), jnp.float32),
            pltpu.VMEM((_RANK, _D), jnp.float32),
            pltpu.VMEM((_D,), jnp.float32),
            pltpu.VMEM((16,), jnp.float32),
            pltpu.VMEM((_D,), jnp.int32),
            pltpu.VMEM((_D,), jnp.float32),
        ],
    )(_sc_kernel)
    out_sc = sc(x[:_SB], wst, b_sel, ut, vr, wg, bg16)

    b_sel2 = b_sel.reshape(_D, 1)
    wg2 = W_gate.reshape(_D, 1)
    b_gate2 = b_gate.reshape(1, 1)
    grid = ((_B - _SB) // _BM,)
    out_tc = pl.pallas_call(
        _tc_kernel,
        grid=grid,
        in_specs=[
            pl.BlockSpec((_BM, _D), lambda i: (i, 0)),
            pl.BlockSpec((_D, _D), lambda i: (0, 0)),
            pl.BlockSpec((_D, 1), lambda i: (0, 0)),
            pl.BlockSpec((_D, _RANK), lambda i: (0, 0)),
            pl.BlockSpec((_RANK, _D), lambda i: (0, 0)),
            pl.BlockSpec((_D, 1), lambda i: (0, 0)),
            pl.BlockSpec((1, 1), lambda i: (0, 0)),
        ],
        out_specs=pl.BlockSpec((_BM, _D), lambda i: (i, 0)),
        out_shape=jax.ShapeDtypeStruct((_B - _SB, _D), jnp.float32),
        compiler_params=pltpu.CompilerParams(
            dimension_semantics=("parallel",)),
    )(x[_SB:], W_sel, b_sel2, U, V, wg2, b_gate2)

    return jnp.concatenate([out_sc, out_tc], axis=0)


# speedup vs baseline: 4.8296x; 1.1300x over previous
import functools
import jax
import jax.numpy as jnp
from jax import lax
from jax.experimental import pallas as pl
from jax.experimental.pallas import tpu as pltpu
from jax.experimental.pallas import tpu_sc as plsc

_B, _D, _RANK, _K = 16384, 64, 6, 32
_SB = 1024
_NW = 32
_RPW = _SB // _NW
_CH = 32
_BM = 1024
_SCALE = 1.0 / (_RANK ** 0.5)
_INTMIN = -0x80000000


def _bf(v):
    b = lax.bitcast_convert_type(v, jnp.int32)
    b = (b + 0x7FFF + jnp.bitwise_and(lax.shift_right_logical(b, 16), 1))
    b = jnp.bitwise_and(b, -65536)
    return lax.bitcast_convert_type(b, jnp.float32)


def _sc_kernel(x_hbm, wst_hbm, bsel_hbm, ut_hbm, v_hbm, wg_hbm, bg_hbm,
               out_hbm, xbuf, obuf, wst, bsel, ut, vv, wg, bg, kbuf, xrow):
    wid = lax.axis_index("s") * 2 + lax.axis_index("c")
    base = wid * _RPW
    pltpu.sync_copy(wst_hbm, wst)
    pltpu.sync_copy(bsel_hbm, bsel)
    pltpu.sync_copy(ut_hbm, ut)
    pltpu.sync_copy(v_hbm, vv)
    pltpu.sync_copy(wg_hbm, wg)
    pltpu.sync_copy(bg_hbm, bg)

    lane = lax.iota(jnp.int32, 16)
    idx_c = [lane + 16 * c for c in range(4)]

    def row_body(r, carry):
        xv = [xbuf[r, pl.ds(16 * c, 16)] for c in range(4)]
        xr = [_bf(xv[c]) for c in range(4)]
        for c in range(4):
            xrow[pl.ds(16 * c, 16)] = xr[c]

        def sel_body(i, z):
            i16 = jnp.full((16,), i, jnp.int32)
            bxi = plsc.load_gather(xrow, [i16])
            return tuple(z[c] + bxi * wst[i, pl.ds(16 * c, 16)]
                         for c in range(4))

        z0 = tuple(bsel[pl.ds(16 * c, 16)] for c in range(4))
        z = lax.fori_loop(0, _D, sel_body, z0)

        kint = []
        for c in range(4):
            b = lax.bitcast_convert_type(z[c], jnp.int32)
            kc = jnp.where(b >= 0, b, _INTMIN - b)
            kbuf[pl.ds(16 * c, 16)] = kc
            kint.append(kc)

        kadj0 = tuple(kint[c] - (idx_c[c] > 0).astype(jnp.int32)
                      for c in range(4))
        rank0 = tuple(jnp.zeros((16,), jnp.int32) for _ in range(4))

        def rank_body(i, kr):
            kadj, rank = kr
            i16 = jnp.full((16,), i, jnp.int32)
            bki = plsc.load_gather(kbuf, [i16])
            rank = tuple(rank[c] + (bki > kadj[c]).astype(jnp.int32)
                         for c in range(4))
            kadj = tuple(kadj[c] + (idx_c[c] == i + 1).astype(jnp.int32)
                         for c in range(4))
            return kadj, rank

        _, rank = lax.fori_loop(0, _D, rank_body, (kadj0, rank0))

        xs = [jnp.where(rank[c] < _K, xv[c], 0.0) for c in range(4)]
        xsr = [jnp.where(rank[c] < _K, xr[c], 0.0) for c in range(4)]

        t = []
        for p in range(_RANK):
            acc = xsr[0] * ut[p, pl.ds(0, 16)]
            for c in range(1, 4):
                acc = acc + xsr[c] * ut[p, pl.ds(16 * c, 16)]
            t.append(_bf(jnp.broadcast_to(jnp.sum(acc), (16,))))
        cross = []
        for c in range(4):
            acc = t[0] * vv[0, pl.ds(16 * c, 16)]
            for p in range(1, _RANK):
                acc = acc + t[p] * vv[p, pl.ds(16 * c, 16)]
            cross.append(acc)
        xi = [xs[c] + _SCALE * (xs[c] * cross[c]) for c in range(4)]

        gacc = _bf(xi[0]) * wg[pl.ds(0, 16)]
        for c in range(1, 4):
            gacc = gacc + _bf(xi[c]) * wg[pl.ds(16 * c, 16)]
        gz = jnp.broadcast_to(jnp.sum(gacc), (16,)) + bg[...]
        g = 1.0 / (1.0 + jnp.exp(-gz))

        for c in range(4):
            obuf[r, pl.ds(16 * c, 16)] = g * xi[c] + (1.0 - g) * xs[c]
        return carry

    for ch in range(_RPW // _CH):
        cbase = base + ch * _CH
        pltpu.sync_copy(x_hbm.at[pl.ds(cbase, _CH)], xbuf)
        lax.fori_loop(0, _CH, row_body, 0)
        pltpu.sync_copy(obuf, out_hbm.at[pl.ds(cbase, _CH)])


def _tc_kernel(x_ref, wsel_ref, bsel_ref, u_ref, v_ref, wg_ref, bg_ref,
               out_ref):
    x = x_ref[...]
    xt = x.T
    zt = jax.lax.dot_general(wsel_ref[...], x, (((1,), (1,)), ((), ())),
                             preferred_element_type=jnp.float32)
    zt = zt + bsel_ref[...]

    row = jax.lax.broadcasted_iota(jnp.int32, (_D, _BM), 0)
    b = zt.view(jnp.int32)
    k = jnp.where(b >= 0, b, jnp.int32(_INTMIN) - b)

    kadj = jnp.where(row > 0, k - 1, k)
    rank = jnp.zeros((_D, _BM), jnp.int32)
    for i in range(_D):
        if i > 0:
            onehot = (row == i).astype(jnp.int32)
            kadj = kadj + onehot
        rank = rank + (kadj[i:i + 1, :] > kadj).astype(jnp.int32)
    xs = jnp.where(rank < _K, xt, 0.0)

    m = jnp.dot(u_ref[...], v_ref[...], preferred_element_type=jnp.float32)
    crosst = jax.lax.dot_general(m, xs, (((0,), (0,)), ((), ())),
                                 preferred_element_type=jnp.float32)
    xi = xs * (1.0 + _SCALE * crosst)

    g = jax.nn.sigmoid(
        jnp.sum(xi * wg_ref[...], axis=0, keepdims=True) + bg_ref[...])
    out_ref[...] = (g * xi + (1.0 - g) * xs).T


def kernel(x, W_sel, b_sel, U, V, W_gate, b_gate):
    bf = lambda a: a.astype(jnp.bfloat16).astype(jnp.float32)
    wst = bf(W_sel.T)
    ut = bf(U.T)
    vr = bf(V)
    wg = bf(W_gate.reshape(_D))
    bg16 = jnp.broadcast_to(b_gate.reshape(1), (16,))
    mesh = plsc.VectorSubcoreMesh(core_axis_name="c", subcore_axis_name="s")
    sc = functools.partial(
        pl.kernel,
        mesh=mesh,
        compiler_params=pltpu.CompilerParams(needs_layout_passes=False),
        out_type=jax.ShapeDtypeStruct((_SB, _D), jnp.float32),
        scratch_types=[
            pltpu.VMEM((_CH, _D), jnp.float32),
            pltpu.VMEM((_CH, _D), jnp.float32),
            pltpu.VMEM((_D, _D), jnp.float32),
            pltpu.VMEM((_D,), jnp.float32),
            pltpu.VMEM((_RANK, _D), jnp.float32),
            pltpu.VMEM((_RANK, _D), jnp.float32),
            pltpu.VMEM((_D,), jnp.float32),
            pltpu.VMEM((16,), jnp.float32),
            pltpu.VMEM((_D,), jnp.int32),
            pltpu.VMEM((_D,), jnp.float32),
        ],
    )(_sc_kernel)
    out_sc = sc(x[:_SB], wst, b_sel, ut, vr, wg, bg16)

    b_sel2 = b_sel.reshape(_D, 1)
    wg2 = W_gate.reshape(_D, 1)
    b_gate2 = b_gate.reshape(1, 1)
    grid = ((_B - _SB) // _BM,)
    out_tc = pl.pallas_call(
        _tc_kernel,
        grid=grid,
        in_specs=[
            pl.BlockSpec((_BM, _D), lambda i: (i, 0)),
            pl.BlockSpec((_D, _D), lambda i: (0, 0)),
            pl.BlockSpec((_D, 1), lambda i: (0, 0)),
            pl.BlockSpec((_D, _RANK), lambda i: (0, 0)),
            pl.BlockSpec((_RANK, _D), lambda i: (0, 0)),
            pl.BlockSpec((_D, 1), lambda i: (0, 0)),
            pl.BlockSpec((1, 1), lambda i: (0, 0)),
        ],
        out_specs=pl.BlockSpec((_BM, _D), lambda i: (i, 0)),
        out_shape=jax.ShapeDtypeStruct((_B - _SB, _D), jnp.float32),
        compiler_params=pltpu.CompilerParams(
            dimension_semantics=("parallel",)),
    )(x[_SB:], W_sel, b_sel2, U, V, wg2, b_gate2)

    return jnp.concatenate([out_sc, out_tc], axis=0)
